# baseline (speedup 1.0000x reference)
import dataclasses
import functools

import jax
import jax.numpy as jnp
from jax import lax
from jax.experimental import pallas as pl
from jax.experimental.pallas import tpu as pltpu
from jax.experimental.pallas import tpu_sc as plsc

N_NODES = 10000
N_EDGES = 320000
D_IN = 128
D_HID = 512
D_OUT = 128

NC = 2
NS = 16
NW = NC * NS
E_PER_W = N_EDGES // NW
BLK = 80
NBLK = E_PER_W // BLK
CHUNK = 25
NCHUNK = NBLK // CHUNK
N_PAD = 10240
STRIPE = N_PAD // NS

_mesh = plsc.VectorSubcoreMesh(core_axis_name="c", subcore_axis_name="s")

_cp = pltpu.CompilerParams()
if "needs_layout_passes" in pltpu.CompilerParams.__dataclass_fields__:
    _cp = dataclasses.replace(_cp, needs_layout_passes=False)


def _bcast_lane(v16, lane):
    idx = jnp.full((16, 1), lane, dtype=jnp.int32)
    dn = lax.GatherDimensionNumbers(
        offset_dims=(), collapsed_slice_dims=(0,), start_index_map=(0,))
    return lax.gather(v16, idx, dn, (1,),
                      mode=lax.GatherScatterMode.PROMISE_IN_BOUNDS)


@functools.partial(
    pl.kernel,
    out_type=jax.ShapeDtypeStruct((NC, N_PAD), jnp.float32),
    mesh=_mesh,
    scratch_types=[
        pltpu.VMEM((NCHUNK, CHUNK, BLK), jnp.int32),
        pltpu.VMEM((NCHUNK, CHUNK, BLK), jnp.float32),
        pltpu.VMEM_SHARED((N_PAD,), jnp.float32),
    ],
    compiler_params=_cp,
)
def _deg_kernel(zeros_hbm, col_hbm, ew_hbm, deg_hbm, col_v, ew_v, deg_sp):
    c = lax.axis_index("c")
    s = lax.axis_index("s")
    pltpu.sync_copy(col_hbm.at[c, s], col_v)
    pltpu.sync_copy(ew_hbm.at[c, s], ew_v)
    pltpu.sync_copy(zeros_hbm, deg_sp.at[pl.ds(s * STRIPE, STRIPE)])
    plsc.subcore_barrier()

    @pl.loop(0, NCHUNK)
    def _ch(q):
        @pl.loop(0, CHUNK)
        def _blk(j):
            pltpu.sync_copy(ew_v.at[q, j], deg_sp.at[col_v.at[q, j]],
                            add=True)

    plsc.subcore_barrier()
    pltpu.sync_copy(deg_sp.at[pl.ds(s * STRIPE, STRIPE)],
                    deg_hbm.at[c].at[pl.ds(s * STRIPE, STRIPE)])


def _newton_rsqrt(d):
    i = plsc.bitcast(d, jnp.int32)
    i = jnp.int32(0x5F3759DF) - lax.shift_right_logical(i, 1)
    y = plsc.bitcast(i, jnp.float32)
    for _ in range(3):
        y = y * (1.5 - 0.5 * d * y * y)
    return y


@functools.partial(
    pl.kernel,
    out_type=[
        jax.ShapeDtypeStruct((NC, N_PAD, D_IN), jnp.float32),
        jax.ShapeDtypeStruct((NC, N_PAD, D_IN), jnp.float32),
        jax.ShapeDtypeStruct((NC, N_PAD), jnp.float32),
    ],
    mesh=_mesh,
    scratch_types=[
        pltpu.VMEM((CHUNK, BLK), jnp.int32),
        pltpu.VMEM((CHUNK, BLK), jnp.int32),
        pltpu.VMEM((CHUNK, BLK), jnp.float32),
        pltpu.VMEM((3, BLK, D_IN), jnp.float32),
        pltpu.VMEM((STRIPE,), jnp.float32),
        pltpu.VMEM((STRIPE,), jnp.float32),
        pltpu.VMEM_SHARED((N_PAD, D_IN), jnp.float32),
        pltpu.SemaphoreType.DMA((3,)),
        pltpu.SemaphoreType.DMA((3,)),
    ],
    compiler_params=_cp,
)
def _agg_kernel(zeros_hbm, x_hbm, deg_hbm, row_hbm, col_hbm, ew_hbm,
                y_hbm, xs_hbm2, dis_hbm,
                row_v, col_v, ew_v, gbuf, dp0, dp1, acc_sp, gsem, ssem):
    c = lax.axis_index("c")
    s = lax.axis_index("s")
    pltpu.sync_copy(zeros_hbm, acc_sp.at[pl.ds(s * STRIPE, STRIPE)])

    pltpu.sync_copy(deg_hbm.at[0].at[pl.ds(s * STRIPE, STRIPE)], dp0)
    pltpu.sync_copy(deg_hbm.at[1].at[pl.ds(s * STRIPE, STRIPE)], dp1)
    for t in range(STRIPE // 16):
        sl = pl.ds(t * 16, 16)
        dp0[sl] = _newton_rsqrt(dp0[sl] + dp1[sl] + 1.0)
    pltpu.sync_copy(dp0, dis_hbm.at[c].at[pl.ds(s * STRIPE, STRIPE)])

    xs_hbm = xs_hbm2.at[c]
    for t in range(STRIPE // BLK):
        base = s * STRIPE + t * BLK

        @pl.when(base < N_NODES)
        def _():
            pltpu.sync_copy(x_hbm.at[pl.ds(base, BLK)], gbuf.at[0])

            @pl.loop(0, BLK // 16)
            def _grp(g):
                w16 = dp0[pl.ds(t * BLK + g * 16, 16)]
                for e in range(16):
                    we = _bcast_lane(w16, e)
                    r = g * 16 + e
                    for k in range(D_IN // 16):
                        sl = (0, r, pl.ds(k * 16, 16))
                        gbuf[sl] = gbuf[sl] * we

            pltpu.sync_copy(gbuf.at[0], xs_hbm.at[pl.ds(base, BLK)])

    plsc.subcore_barrier()

    def g_start(j, b):
        pltpu.async_copy(xs_hbm.at[row_v.at[j]], gbuf.at[b], gsem.at[b])

    def g_wait(j, b):
        pltpu.make_async_copy(xs_hbm.at[row_v.at[j]], gbuf.at[b],
                              gsem.at[b]).wait()

    def s_start(j, b):
        pltpu.async_copy(gbuf.at[b], acc_sp.at[col_v.at[j]], ssem.at[b],
                         add=True)

    def s_wait(j, b):
        pltpu.make_async_copy(gbuf.at[b], acc_sp.at[col_v.at[j]],
                              ssem.at[b]).wait()

    def scale(j, b):
        @pl.loop(0, BLK // 16)
        def _grp(g):
            w16 = ew_v[j, pl.ds(g * 16, 16)]
            for e in range(16):
                we = _bcast_lane(w16, e)
                r = g * 16 + e
                for k in range(D_IN // 16):
                    sl = (b, r, pl.ds(k * 16, 16))
                    gbuf[sl] = gbuf[sl] * we

    @pl.loop(0, NCHUNK)
    def _ch(q):
        pltpu.sync_copy(row_hbm.at[c, s, q], row_v)
        pltpu.sync_copy(col_hbm.at[c, s, q], col_v)
        pltpu.sync_copy(ew_hbm.at[c, s, q], ew_v)

        g_start(0, 0)
        g_start(1, 1)
        g_wait(0, 0)
        scale(0, 0)
        s_start(0, 0)
        g_start(2, 2)
        g_wait(1, 1)
        scale(1, 1)
        s_start(1, 1)

        @pl.loop(0, (CHUNK - 4) // 3)
        def _t(t):
            base = t * 3 + 2
            for d in range(3):
                j = base + d
                b = (2 + d) % 3
                s_wait(j - 2, d)
                g_start(j + 1, d)
                g_wait(j, b)
                scale(j, b)
                s_start(j, b)

        s_wait(CHUNK - 4, 0)
        g_start(CHUNK - 1, 0)
        g_wait(CHUNK - 2, 2)
        scale(CHUNK - 2, 2)
        s_start(CHUNK - 2, 2)
        s_wait(CHUNK - 3, 1)
        g_wait(CHUNK - 1, 0)
        scale(CHUNK - 1, 0)
        s_start(CHUNK - 1, 0)
        s_wait(CHUNK - 2, 2)
        s_wait(CHUNK - 1, 0)

    plsc.subcore_barrier()
    pltpu.sync_copy(acc_sp.at[pl.ds(s * STRIPE, STRIPE)],
                    y_hbm.at[c].at[pl.ds(s * STRIPE, STRIPE)])


def _mlp_body(dis_ref, y_ref, xs_ref, wc_ref, bc_ref, wl_ref, bl_ref, o_ref):
    z = (y_ref[0] + y_ref[1] + xs_ref[0]) * dis_ref[0]
    h = jnp.maximum(
        jnp.dot(z, wc_ref[...], precision=lax.Precision.HIGHEST,
                preferred_element_type=jnp.float32) + bc_ref[...], 0.0)
    o_ref[...] = jnp.maximum(
        jnp.dot(h, wl_ref[...], precision=lax.Precision.HIGHEST,
                preferred_element_type=jnp.float32) + bl_ref[...], 0.0)


ROWS_B = 1000


def kernel(x, edge_index, edge_weight, W_conv, b_conv, W_lin, b_lin):
    row_r = edge_index[0].astype(jnp.int32).reshape(NC, NS, NCHUNK, CHUNK, BLK)
    col_r = edge_index[1].astype(jnp.int32).reshape(NC, NS, NCHUNK, CHUNK, BLK)
    ew_r = edge_weight.reshape(NC, NS, NCHUNK, CHUNK, BLK)
    zeros1 = jnp.zeros((STRIPE,), jnp.float32)
    zeros2 = jnp.zeros((STRIPE, D_IN), jnp.float32)

    deg_parts = _deg_kernel(zeros1, col_r, ew_r)

    y_parts, xs_parts, dis_parts = _agg_kernel(
        zeros2, x, deg_parts, row_r, col_r, ew_r)
    dis3 = dis_parts.reshape(NC, N_PAD, 1)

    out = pl.pallas_call(
        _mlp_body,
        grid=(N_NODES // ROWS_B,),
        in_specs=[
            pl.BlockSpec((1, ROWS_B, 1), lambda i: (0, i, 0)),
            pl.BlockSpec((NC, ROWS_B, D_IN), lambda i: (0, i, 0)),
            pl.BlockSpec((1, ROWS_B, D_IN), lambda i: (0, i, 0)),
            pl.BlockSpec((D_IN, D_HID), lambda i: (0, 0)),
            pl.BlockSpec((1, D_HID), lambda i: (0, 0)),
            pl.BlockSpec((D_HID, D_OUT), lambda i: (0, 0)),
            pl.BlockSpec((1, D_OUT), lambda i: (0, 0)),
        ],
        out_specs=pl.BlockSpec((ROWS_B, D_OUT), lambda i: (i, 0)),
        out_shape=jax.ShapeDtypeStruct((N_NODES, D_OUT), jnp.float32),
    )(dis3, y_parts, xs_parts, W_conv, b_conv.reshape(1, D_HID),
      W_lin, b_lin.reshape(1, D_OUT))
    return out


# speedup vs baseline: 33.7399x; 33.7399x over previous
import dataclasses
import functools

import jax
import jax.numpy as jnp
from jax import lax
from jax.experimental import pallas as pl
from jax.experimental.pallas import tpu as pltpu
from jax.experimental.pallas import tpu_sc as plsc

N_NODES = 10000
N_EDGES = 320000
D_IN = 128
D_HID = 512
D_OUT = 128

NC = 2
NS = 16
NW = NC * NS
E_PER_W = N_EDGES // NW
BLK = 80
NBLK = E_PER_W // BLK
CHUNK = 25
NCHUNK = NBLK // CHUNK
N_PAD = 10240
STRIPE = N_PAD // NS

_mesh = plsc.VectorSubcoreMesh(core_axis_name="c", subcore_axis_name="s")

_cp = pltpu.CompilerParams()
if "needs_layout_passes" in pltpu.CompilerParams.__dataclass_fields__:
    _cp = dataclasses.replace(_cp, needs_layout_passes=False)


def _bcast_lane(v16, lane):
    idx = jnp.full((16, 1), lane, dtype=jnp.int32)
    dn = lax.GatherDimensionNumbers(
        offset_dims=(), collapsed_slice_dims=(0,), start_index_map=(0,))
    return lax.gather(v16, idx, dn, (1,),
                      mode=lax.GatherScatterMode.PROMISE_IN_BOUNDS)


@functools.partial(
    pl.kernel,
    out_type=jax.ShapeDtypeStruct((NC, N_PAD), jnp.float32),
    mesh=_mesh,
    scratch_types=[
        pltpu.VMEM((NCHUNK, CHUNK, BLK), jnp.int32),
        pltpu.VMEM((NCHUNK, CHUNK, BLK), jnp.float32),
        pltpu.VMEM_SHARED((N_PAD,), jnp.float32),
    ],
    compiler_params=_cp,
)
def _deg_kernel(zeros_hbm, ei_hbm, ew_hbm, deg_hbm, col_v, ew_v, deg_sp):
    col_hbm = ei_hbm.at[1]
    c = lax.axis_index("c")
    s = lax.axis_index("s")
    pltpu.sync_copy(col_hbm.at[c, s], col_v)
    pltpu.sync_copy(ew_hbm.at[c, s], ew_v)
    pltpu.sync_copy(zeros_hbm, deg_sp.at[pl.ds(s * STRIPE, STRIPE)])
    plsc.subcore_barrier()

    @pl.loop(0, NCHUNK)
    def _ch(q):
        @pl.loop(0, CHUNK)
        def _blk(j):
            pltpu.sync_copy(ew_v.at[q, j], deg_sp.at[col_v.at[q, j]],
                            add=True)

    plsc.subcore_barrier()
    pltpu.sync_copy(deg_sp.at[pl.ds(s * STRIPE, STRIPE)],
                    deg_hbm.at[c].at[pl.ds(s * STRIPE, STRIPE)])


def _newton_rsqrt(d):
    i = plsc.bitcast(d, jnp.int32)
    i = jnp.int32(0x5F3759DF) - lax.shift_right_logical(i, 1)
    y = plsc.bitcast(i, jnp.float32)
    for _ in range(3):
        y = y * (1.5 - 0.5 * d * y * y)
    return y


@functools.partial(
    pl.kernel,
    out_type=[
        jax.ShapeDtypeStruct((NC, N_PAD, D_IN), jnp.float32),
        jax.ShapeDtypeStruct((NC, N_PAD, D_IN), jnp.float32),
        jax.ShapeDtypeStruct((NC, N_PAD), jnp.float32),
    ],
    mesh=_mesh,
    scratch_types=[
        pltpu.VMEM((CHUNK, BLK), jnp.int32),
        pltpu.VMEM((CHUNK, BLK), jnp.int32),
        pltpu.VMEM((CHUNK, BLK), jnp.float32),
        pltpu.VMEM((3, BLK, D_IN), jnp.float32),
        pltpu.VMEM((STRIPE,), jnp.float32),
        pltpu.VMEM((STRIPE,), jnp.float32),
        pltpu.VMEM_SHARED((N_PAD, D_IN), jnp.float32),
        pltpu.SemaphoreType.DMA((3,)),
        pltpu.SemaphoreType.DMA((3,)),
    ],
    compiler_params=_cp,
)
def _agg_kernel(zeros_hbm, x_hbm, deg_hbm, ei_hbm, ew_hbm,
                y_hbm, xs_hbm2, dis_hbm,
                row_v, col_v, ew_v, gbuf, dp0, dp1, acc_sp, gsem, ssem):
    row_hbm = ei_hbm.at[0]
    col_hbm = ei_hbm.at[1]
    c = lax.axis_index("c")
    s = lax.axis_index("s")
    pltpu.sync_copy(zeros_hbm, acc_sp.at[pl.ds(s * STRIPE, STRIPE)])

    pltpu.sync_copy(deg_hbm.at[0].at[pl.ds(s * STRIPE, STRIPE)], dp0)
    pltpu.sync_copy(deg_hbm.at[1].at[pl.ds(s * STRIPE, STRIPE)], dp1)
    for t in range(STRIPE // 16):
        sl = pl.ds(t * 16, 16)
        dp0[sl] = _newton_rsqrt(dp0[sl] + dp1[sl] + 1.0)
    pltpu.sync_copy(dp0, dis_hbm.at[c].at[pl.ds(s * STRIPE, STRIPE)])

    xs_hbm = xs_hbm2.at[c]
    for t in range(STRIPE // BLK):
        base = s * STRIPE + t * BLK

        @pl.when(base < N_NODES)
        def _():
            pltpu.sync_copy(x_hbm.at[pl.ds(base, BLK)], gbuf.at[0])

            @pl.loop(0, BLK // 16)
            def _grp(g):
                w16 = dp0[pl.ds(t * BLK + g * 16, 16)]
                for e in range(16):
                    we = _bcast_lane(w16, e)
                    r = g * 16 + e
                    for k in range(D_IN // 16):
                        sl = (0, r, pl.ds(k * 16, 16))
                        gbuf[sl] = gbuf[sl] * we

            pltpu.sync_copy(gbuf.at[0], xs_hbm.at[pl.ds(base, BLK)])

    plsc.subcore_barrier()

    def g_start(j, b):
        pltpu.async_copy(xs_hbm.at[row_v.at[j]], gbuf.at[b], gsem.at[b])

    def g_wait(j, b):
        pltpu.make_async_copy(xs_hbm.at[row_v.at[j]], gbuf.at[b],
                              gsem.at[b]).wait()

    def s_start(j, b):
        pltpu.async_copy(gbuf.at[b], acc_sp.at[col_v.at[j]], ssem.at[b],
                         add=True)

    def s_wait(j, b):
        pltpu.make_async_copy(gbuf.at[b], acc_sp.at[col_v.at[j]],
                              ssem.at[b]).wait()

    def scale(j, b):
        @pl.loop(0, BLK // 16)
        def _grp(g):
            w16 = ew_v[j, pl.ds(g * 16, 16)]
            for e in range(16):
                we = _bcast_lane(w16, e)
                r = g * 16 + e
                for k in range(D_IN // 16):
                    sl = (b, r, pl.ds(k * 16, 16))
                    gbuf[sl] = gbuf[sl] * we

    @pl.loop(0, NCHUNK)
    def _ch(q):
        pltpu.sync_copy(row_hbm.at[c, s, q], row_v)
        pltpu.sync_copy(col_hbm.at[c, s, q], col_v)
        pltpu.sync_copy(ew_hbm.at[c, s, q], ew_v)

        g_start(0, 0)
        g_start(1, 1)
        g_wait(0, 0)
        scale(0, 0)
        s_start(0, 0)
        g_start(2, 2)
        g_wait(1, 1)
        scale(1, 1)
        s_start(1, 1)

        @pl.loop(0, (CHUNK - 4) // 3)
        def _t(t):
            base = t * 3 + 2
            for d in range(3):
                j = base + d
                b = (2 + d) % 3
                s_wait(j - 2, d)
                g_start(j + 1, d)
                g_wait(j, b)
                scale(j, b)
                s_start(j, b)

        s_wait(CHUNK - 4, 0)
        g_start(CHUNK - 1, 0)
        g_wait(CHUNK - 2, 2)
        scale(CHUNK - 2, 2)
        s_start(CHUNK - 2, 2)
        s_wait(CHUNK - 3, 1)
        g_wait(CHUNK - 1, 0)
        scale(CHUNK - 1, 0)
        s_start(CHUNK - 1, 0)
        s_wait(CHUNK - 2, 2)
        s_wait(CHUNK - 1, 0)

    plsc.subcore_barrier()
    pltpu.sync_copy(acc_sp.at[pl.ds(s * STRIPE, STRIPE)],
                    y_hbm.at[c].at[pl.ds(s * STRIPE, STRIPE)])


def _mlp_body(dis_ref, y_ref, xs_ref, wc_ref, bc_ref, wl_ref, bl_ref, o_ref):
    z = (y_ref[0] + y_ref[1] + xs_ref[0]) * dis_ref[0]
    h = jnp.maximum(
        jnp.dot(z, wc_ref[...],
                preferred_element_type=jnp.float32) + bc_ref[...], 0.0)
    o_ref[...] = jnp.maximum(
        jnp.dot(h, wl_ref[...],
                preferred_element_type=jnp.float32) + bl_ref[...], 0.0)


ROWS_B = 1000


def kernel(x, edge_index, edge_weight, W_conv, b_conv, W_lin, b_lin):
    ei6 = edge_index.astype(jnp.int32).reshape(2, NC, NS, NCHUNK, CHUNK, BLK)
    ew_r = edge_weight.reshape(NC, NS, NCHUNK, CHUNK, BLK)
    zeros1 = jnp.zeros((STRIPE,), jnp.float32)
    zeros2 = jnp.zeros((STRIPE, D_IN), jnp.float32)

    deg_parts = _deg_kernel(zeros1, ei6, ew_r)

    y_parts, xs_parts, dis_parts = _agg_kernel(
        zeros2, x, deg_parts, ei6, ew_r)
    dis3 = dis_parts.reshape(NC, N_PAD, 1)

    out = pl.pallas_call(
        _mlp_body,
        grid=(N_NODES // ROWS_B,),
        in_specs=[
            pl.BlockSpec((1, ROWS_B, 1), lambda i: (0, i, 0)),
            pl.BlockSpec((NC, ROWS_B, D_IN), lambda i: (0, i, 0)),
            pl.BlockSpec((1, ROWS_B, D_IN), lambda i: (0, i, 0)),
            pl.BlockSpec((D_IN, D_HID), lambda i: (0, 0)),
            pl.BlockSpec((1, D_HID), lambda i: (0, 0)),
            pl.BlockSpec((D_HID, D_OUT), lambda i: (0, 0)),
            pl.BlockSpec((1, D_OUT), lambda i: (0, 0)),
        ],
        out_specs=pl.BlockSpec((ROWS_B, D_OUT), lambda i: (i, 0)),
        out_shape=jax.ShapeDtypeStruct((N_NODES, D_OUT), jnp.float32),
    )(dis3, y_parts, xs_parts, W_conv, b_conv.reshape(1, D_HID),
      W_lin, b_lin.reshape(1, D_OUT))
    return out
